# Initial kernel scaffold; baseline (speedup 1.0000x reference)
import jax
import jax.numpy as jnp
from jax.experimental import pallas as pl


def kernel(output, target):
    raise NotImplementedError("write your pallas kernel here")


# speedup vs baseline: 1.5285x; 1.5285x over previous
import jax
import jax.numpy as jnp
from jax.experimental import pallas as pl

_R = 64
_NSTEPS = 512 // _R


def _dice_body(o_ref, t_ref, inter_ref, card_ref, score_ref):
    step = pl.program_id(0)

    o = o_ref[...]
    t = t_ref[...][:, 0]

    o0, o1, o2, o3 = o[:, 0], o[:, 1], o[:, 2], o[:, 3]
    gt1 = o1 > o0
    gt3 = o3 > o2
    m01 = jnp.maximum(o0, o1)
    m23 = jnp.maximum(o2, o3)
    gtb = m23 > m01
    lo = jnp.where(gt1, 1, 0)
    hi = jnp.where(gt3, 3, 2)
    idx = jnp.where(gtb, hi, lo)

    cls = jax.lax.broadcasted_iota(jnp.int32, (1, 4, 1, 1), 1)
    pm = idx[:, None] == cls
    tm = t[:, None] == cls
    one = jnp.float32(1.0)
    zero = jnp.float32(0.0)
    i_part = jnp.sum(jnp.where(pm & tm, one, zero), axis=(2, 3))
    c_part = (jnp.sum(jnp.where(pm, one, zero), axis=(2, 3))
              + jnp.sum(jnp.where(tm, one, zero), axis=(2, 3)))

    @pl.when(step == 0)
    def _():
        inter_ref[...] = jnp.zeros_like(inter_ref)
        card_ref[...] = jnp.zeros_like(card_ref)

    inter_ref[...] += i_part
    card_ref[...] += c_part

    @pl.when(step == _NSTEPS - 1)
    def _():
        inter = inter_ref[...]
        card = card_ref[...]
        score_ref[...] = jnp.mean(
            2.0 * inter / jnp.maximum(card, 1.0), axis=0, keepdims=True)


def kernel(output, target):
    _, _, score = pl.pallas_call(
        _dice_body,
        grid=(_NSTEPS,),
        in_specs=[
            pl.BlockSpec((8, 4, _R, 512), lambda i: (0, 0, i, 0)),
            pl.BlockSpec((8, 1, _R, 512), lambda i: (0, 0, i, 0)),
        ],
        out_specs=[
            pl.BlockSpec((8, 4), lambda i: (0, 0)),
            pl.BlockSpec((8, 4), lambda i: (0, 0)),
            pl.BlockSpec((1, 4), lambda i: (0, 0)),
        ],
        out_shape=[
            jax.ShapeDtypeStruct((8, 4), jnp.float32),
            jax.ShapeDtypeStruct((8, 4), jnp.float32),
            jax.ShapeDtypeStruct((1, 4), jnp.float32),
        ],
    )(output, target)
    return score[0]


# speedup vs baseline: 1.6343x; 1.0692x over previous
import jax
import jax.numpy as jnp
from jax.experimental import pallas as pl

_R = 64
_NSTEPS = 512 // _R
_NPIX = float(512 * 512)


def _dice_body(o_ref, t_ref, stats_ref, score_ref):
    step = pl.program_id(0)

    o = o_ref[...]
    t = t_ref[...][:, 0]

    o0, o1, o2, o3 = o[:, 0], o[:, 1], o[:, 2], o[:, 3]
    gt1 = o1 > o0
    gt3 = o3 > o2
    gtb = jnp.maximum(o2, o3) > jnp.maximum(o0, o1)
    one = jnp.int32(1)
    zero = jnp.int32(0)
    a = jnp.where(gtb, one, zero)
    b = jnp.where(gtb, jnp.where(gt3, one, zero),
                  jnp.where(gt1, one, zero))
    ta = t >> 1
    tb = t & 1
    ab = a & b
    tab = ta & tb
    m = ((a ^ ta) | (b ^ tb)) ^ 1
    ma = m & a
    mb = m & b
    mab = ma & b

    def s(x):
        return jnp.sum(x, axis=(1, 2))

    part = jnp.stack(
        [s(a), s(b), s(ab), s(ta), s(tb), s(tab), s(m), s(ma), s(mb), s(mab)],
        axis=0)

    @pl.when(step == 0)
    def _():
        stats_ref[...] = jnp.zeros_like(stats_ref)

    stats_ref[...] += part

    @pl.when(step == _NSTEPS - 1)
    def _():
        st = stats_ref[...].astype(jnp.float32)
        sa, sb, sab = st[0], st[1], st[2]
        sta, stb, stab = st[3], st[4], st[5]
        sm, sma, smb, smab = st[6], st[7], st[8], st[9]
        p3, p2, p1 = sab, sa - sab, sb - sab
        p0 = _NPIX - sa - sb + sab
        t3, t2, t1 = stab, sta - stab, stb - stab
        t0 = _NPIX - sta - stb + stab
        i3, i2, i1 = smab, sma - smab, smb - smab
        i0 = sm - sma - smb + smab
        inter = jnp.stack([i0, i1, i2, i3], axis=1)
        card = (jnp.stack([p0, p1, p2, p3], axis=1)
                + jnp.stack([t0, t1, t2, t3], axis=1))
        score_ref[...] = jnp.mean(
            2.0 * inter / jnp.maximum(card, 1.0), axis=0, keepdims=True)


def kernel(output, target):
    _, score = pl.pallas_call(
        _dice_body,
        grid=(_NSTEPS,),
        in_specs=[
            pl.BlockSpec((8, 4, _R, 512), lambda i: (0, 0, i, 0)),
            pl.BlockSpec((8, 1, _R, 512), lambda i: (0, 0, i, 0)),
        ],
        out_specs=[
            pl.BlockSpec((10, 8), lambda i: (0, 0)),
            pl.BlockSpec((1, 4), lambda i: (0, 0)),
        ],
        out_shape=[
            jax.ShapeDtypeStruct((10, 8), jnp.int32),
            jax.ShapeDtypeStruct((1, 4), jnp.float32),
        ],
    )(output, target)
    return score[0]
